# baseline (speedup 1.0000x reference)
import functools

import jax
import jax.numpy as jnp
from jax import lax
from jax.experimental import pallas as pl
from jax.experimental.pallas import tpu as pltpu
from jax.experimental.pallas import tpu_sc as plsc

B = 4096
D = 64
NW = 32
BPW = B // NW
NBUF = 8


def _sc_gather_body(idx_hbm, iet_hbm, out_hbm, idx_v, blk_v, rows_v, sems):
  wid = lax.axis_index("s") * 2 + lax.axis_index("c")
  base = wid * BPW
  pltpu.sync_copy(idx_hbm.at[pl.ds(base, BPW)], idx_v)

  def chunk(c, carry):
    vec = idx_v[pl.ds(c * 16, 16)]
    for w in range(16 // NBUF):
      descs = []
      for b in range(NBUF):
        idx = vec[w * NBUF + b]
        loff = pl.multiple_of((idx >> 7) * 128, 128)
        d = pltpu.make_async_copy(
            iet_hbm.at[:, pl.ds(loff, 128)],
            blk_v.at[b],
            sems.at[b],
        )
        d.start()
        descs.append(d)
      for b in range(NBUF):
        descs[b].wait()
        idx = vec[w * NBUF + b]
        m = jnp.broadcast_to(idx & 127, (16,))
        k = jnp.broadcast_to(c * 16 + w * NBUF + b, (16,))
        for s in range(D // 16):
          rows = lax.broadcasted_iota(jnp.int32, (16,), 0) + (16 * s)
          col = plsc.load_gather(blk_v.at[b], [rows, m])
          plsc.store_scatter(rows_v, [rows, k], col)
    return carry

  lax.fori_loop(0, BPW // 16, chunk, 0)
  pltpu.sync_copy(rows_v, out_hbm.at[:, pl.ds(pl.multiple_of(base, 128), BPW)])


_sc_gather = functools.partial(
    pl.kernel,
    out_type=jax.ShapeDtypeStruct((D, B), jnp.float32),
    mesh=plsc.VectorSubcoreMesh(core_axis_name="c", subcore_axis_name="s"),
    scratch_types=[
        pltpu.VMEM((BPW,), jnp.int32),
        pltpu.VMEM((NBUF, D, 128), jnp.float32),
        pltpu.VMEM((D, BPW), jnp.float32),
        pltpu.SemaphoreType.DMA((NBUF,)),
    ],
    compiler_params=pltpu.CompilerParams(disable_bounds_checks=True,
                                         needs_layout_passes=False),
)(_sc_gather_body)


GI = 128


def _tc_gather_body(idx_ref, *refs):
  blocks, out = refs[:GI], refs[GI]
  i = pl.program_id(0)
  lane = lax.broadcasted_iota(jnp.int32, (1, 128), 1)
  cols = []
  for j in range(GI):
    m = idx_ref[i * GI + j] & 127
    blk = blocks[j][...]
    cols.append(jnp.sum(jnp.where(lane == m, blk, 0.0), axis=1,
                        keepdims=True))
  out[...] = jnp.concatenate(cols, axis=1)


def _tc_block_map(j, i, idx_ref):
  return (0, idx_ref[i * GI + j] >> 7)


def _tc_gather(idx_arr, table_t):
  return pl.pallas_call(
      _tc_gather_body,
      grid_spec=pltpu.PrefetchScalarGridSpec(
          num_scalar_prefetch=1,
          grid=(B // GI,),
          in_specs=[
              pl.BlockSpec((D, 128), functools.partial(_tc_block_map, j))
              for j in range(GI)
          ],
          out_specs=pl.BlockSpec((D, GI), lambda i, idx_ref: (0, i)),
      ),
      out_shape=jax.ShapeDtypeStruct((D, B), jnp.float32),
  )(idx_arr, *([table_t] * GI))


BM = 512


def _score_body(u_ref, it_ref, o_ref):
  dn = (((0,), (0,)), ((), ()))
  o_ref[...] = lax.dot_general(u_ref[...], it_ref[...], dn,
                               preferred_element_type=jnp.float32)


def _score(u_t, item_t):
  return pl.pallas_call(
      _score_body,
      grid=(B // BM,),
      in_specs=[
          pl.BlockSpec((D, BM), lambda i: (0, i)),
          pl.BlockSpec((D, B), lambda i: (0, 0)),
      ],
      out_specs=pl.BlockSpec((BM, B), lambda i: (i, 0)),
      out_shape=jax.ShapeDtypeStruct((B, B), jnp.float32),
  )(u_t, item_t)


@jax.jit
def kernel(user, pos_item, neg_item, user_emb, item_emb):
  iet = item_emb.T
  item_i_t = _sc_gather(pos_item, iet)
  item_j_t = _sc_gather(neg_item, iet)
  u_t = _tc_gather(user, user_emb.T)
  return (_score(u_t, item_i_t), _score(u_t, item_j_t))


# speedup vs baseline: 1.0059x; 1.0059x over previous
import functools

import jax
import jax.numpy as jnp
from jax import lax
from jax.experimental import pallas as pl
from jax.experimental.pallas import tpu as pltpu
from jax.experimental.pallas import tpu_sc as plsc

B = 4096
D = 64
NW = 32
BPW = B // NW
NBUF = 8


def _sc_gather_body(idx_hbm, iet_hbm, out_hbm, idx_v, blk_v, rows_v, sems):
  wid = lax.axis_index("s") * 2 + lax.axis_index("c")
  base = wid * BPW
  pltpu.sync_copy(idx_hbm.at[pl.ds(base, BPW)], idx_v)

  def chunk(c, carry):
    vec = idx_v[pl.ds(c * 16, 16)]
    for w in range(16 // NBUF):
      descs = []
      for b in range(NBUF):
        idx = vec[w * NBUF + b]
        loff = pl.multiple_of((idx >> 7) * 128, 128)
        d = pltpu.make_async_copy(
            iet_hbm.at[:, pl.ds(loff, 128)],
            blk_v.at[b],
            sems.at[b],
        )
        d.start()
        descs.append(d)
      for b in range(NBUF):
        descs[b].wait()
        idx = vec[w * NBUF + b]
        m = jnp.broadcast_to(idx & 127, (16,))
        k = jnp.broadcast_to(c * 16 + w * NBUF + b, (16,))
        for s in range(D // 16):
          rows = lax.broadcasted_iota(jnp.int32, (16,), 0) + (16 * s)
          col = plsc.load_gather(blk_v.at[b], [rows, m])
          plsc.store_scatter(rows_v, [rows, k], col)
    return carry

  lax.fori_loop(0, BPW // 16, chunk, 0)
  pltpu.sync_copy(rows_v, out_hbm.at[:, pl.ds(pl.multiple_of(base, 128), BPW)])


_sc_gather = functools.partial(
    pl.kernel,
    out_type=jax.ShapeDtypeStruct((D, B), jnp.float32),
    mesh=plsc.VectorSubcoreMesh(core_axis_name="c", subcore_axis_name="s"),
    scratch_types=[
        pltpu.VMEM((BPW,), jnp.int32),
        pltpu.VMEM((NBUF, D, 128), jnp.float32),
        pltpu.VMEM((D, BPW), jnp.float32),
        pltpu.SemaphoreType.DMA((NBUF,)),
    ],
    compiler_params=pltpu.CompilerParams(disable_bounds_checks=True,
                                         needs_layout_passes=False),
)(_sc_gather_body)


GI = 128


def _tc_gather_body(idx_ref, *refs):
  blocks, out = refs[:GI], refs[GI]
  i = pl.program_id(0)
  lane = lax.broadcasted_iota(jnp.int32, (1, 128), 1)
  cols = []
  for j in range(GI):
    m = idx_ref[i * GI + j] & 127
    blk = blocks[j][...]
    cols.append(jnp.sum(jnp.where(lane == m, blk, 0.0), axis=1,
                        keepdims=True))
  out[...] = jnp.concatenate(cols, axis=1)


def _tc_block_map(j, i, idx_ref):
  return (0, idx_ref[i * GI + j] >> 7)


def _tc_gather(idx_arr, table_t):
  return pl.pallas_call(
      _tc_gather_body,
      grid_spec=pltpu.PrefetchScalarGridSpec(
          num_scalar_prefetch=1,
          grid=(B // GI,),
          in_specs=[
              pl.BlockSpec((D, 128), functools.partial(_tc_block_map, j))
              for j in range(GI)
          ],
          out_specs=pl.BlockSpec((D, GI), lambda i, idx_ref: (0, i)),
      ),
      out_shape=jax.ShapeDtypeStruct((D, B), jnp.float32),
  )(idx_arr, *([table_t] * GI))


BM = 512


def _score_body(u_ref, it_ref, o_ref):
  dn = (((0,), (0,)), ((), ()))
  o_ref[...] = lax.dot_general(u_ref[...], it_ref[...], dn,
                               preferred_element_type=jnp.float32)


def _score(u_t, item_t):
  return pl.pallas_call(
      _score_body,
      grid=(B // BM,),
      in_specs=[
          pl.BlockSpec((D, BM), lambda i: (0, i)),
          pl.BlockSpec((D, B), lambda i: (0, 0)),
      ],
      out_specs=pl.BlockSpec((BM, B), lambda i: (i, 0)),
      out_shape=jax.ShapeDtypeStruct((B, B), jnp.float32),
  )(u_t, item_t)


@jax.jit
def kernel(user, pos_item, neg_item, user_emb, item_emb):
  iet = item_emb.T
  item_i_t = _sc_gather(pos_item, iet)
  u_t = _tc_gather(user, user_emb.T)
  pred_i = _score(u_t, item_i_t)
  item_j_t = _sc_gather(neg_item, iet)
  pred_j = _score(u_t, item_j_t)
  return (pred_i, pred_j)
